# Initial kernel scaffold; baseline (speedup 1.0000x reference)
import jax
import jax.numpy as jnp
from jax.experimental import pallas as pl


def kernel(x, gate_w, gate_b):
    raise NotImplementedError("write your pallas kernel here")


# speedup vs baseline: 2.3234x; 2.3234x over previous
import jax
import jax.numpy as jnp
from jax.experimental import pallas as pl

EXPERTS = 16
HIDDEN = 2048
TOKENS = 8192
BLOCK = 512


def _gating_body(x_ref, w_ref, b_ref, sparse_ref, idx_ref, logits_ref):
    logits = jax.lax.dot_general(
        x_ref[:], w_ref[:], (((1,), (1,)), ((), ())),
        preferred_element_type=jnp.float32,
    ) + b_ref[:]
    logits_ref[:] = logits

    col = jax.lax.broadcasted_iota(jnp.int32, logits.shape, 1)
    m1 = jnp.max(logits, axis=1, keepdims=True)
    i1 = jnp.min(jnp.where(logits == m1, col, EXPERTS), axis=1, keepdims=True)
    masked = jnp.where(col == i1, -jnp.inf, logits)
    m2 = jnp.max(masked, axis=1, keepdims=True)
    i2 = jnp.min(jnp.where(masked == m2, col, EXPERTS), axis=1, keepdims=True)

    e2 = jnp.exp(m2 - m1)
    denom = 1.0 + e2
    sparse_ref[:] = jnp.where(col == i1, 1.0 / denom,
                              jnp.where(col == i2, e2 / denom, 0.0))
    idx_ref[:] = jnp.concatenate([i1, i2], axis=1)


def kernel(x, gate_w, gate_b):
    grid = (TOKENS // BLOCK,)
    sparse, idx, logits = pl.pallas_call(
        _gating_body,
        grid=grid,
        in_specs=[
            pl.BlockSpec((BLOCK, HIDDEN), lambda i: (i, 0)),
            pl.BlockSpec((EXPERTS, HIDDEN), lambda i: (0, 0)),
            pl.BlockSpec((1, EXPERTS), lambda i: (0, 0)),
        ],
        out_specs=[
            pl.BlockSpec((BLOCK, EXPERTS), lambda i: (i, 0)),
            pl.BlockSpec((BLOCK, 2), lambda i: (i, 0)),
            pl.BlockSpec((BLOCK, EXPERTS), lambda i: (i, 0)),
        ],
        out_shape=[
            jax.ShapeDtypeStruct((TOKENS, EXPERTS), jnp.float32),
            jax.ShapeDtypeStruct((TOKENS, 2), jnp.int32),
            jax.ShapeDtypeStruct((TOKENS, EXPERTS), jnp.float32),
        ],
    )(x, gate_w, gate_b.reshape(1, EXPERTS))
    return (sparse, idx, logits)


# speedup vs baseline: 2.6097x; 1.1232x over previous
import jax
import jax.numpy as jnp
from jax.experimental import pallas as pl

EXPERTS = 16
HIDDEN = 2048
TOKENS = 8192
BLOCK = 1024


def _gating_body(x_ref, w_ref, b_ref, sparse_ref, idx_ref, logits_ref):
    logits = jax.lax.dot_general(
        x_ref[:], w_ref[:], (((1,), (1,)), ((), ())),
        preferred_element_type=jnp.float32,
    ) + b_ref[:]
    logits_ref[:] = logits

    col = jax.lax.broadcasted_iota(jnp.int32, logits.shape, 1)
    m1 = jnp.max(logits, axis=1, keepdims=True)
    i1 = jnp.min(jnp.where(logits == m1, col, EXPERTS), axis=1, keepdims=True)
    masked = jnp.where(col == i1, -jnp.inf, logits)
    m2 = jnp.max(masked, axis=1, keepdims=True)
    i2 = jnp.min(jnp.where(masked == m2, col, EXPERTS), axis=1, keepdims=True)

    e2 = jnp.exp(m2 - m1)
    denom = 1.0 + e2
    sparse_ref[:] = jnp.where(col == i1, 1.0 / denom,
                              jnp.where(col == i2, e2 / denom, 0.0))
    idx_ref[:] = jnp.concatenate([i1, i2], axis=1)


def kernel(x, gate_w, gate_b):
    grid = (TOKENS // BLOCK,)
    sparse, idx, logits = pl.pallas_call(
        _gating_body,
        grid=grid,
        in_specs=[
            pl.BlockSpec((BLOCK, HIDDEN), lambda i: (i, 0)),
            pl.BlockSpec((EXPERTS, HIDDEN), lambda i: (0, 0)),
            pl.BlockSpec((1, EXPERTS), lambda i: (0, 0)),
        ],
        out_specs=[
            pl.BlockSpec((BLOCK, EXPERTS), lambda i: (i, 0)),
            pl.BlockSpec((BLOCK, 2), lambda i: (i, 0)),
            pl.BlockSpec((BLOCK, EXPERTS), lambda i: (i, 0)),
        ],
        out_shape=[
            jax.ShapeDtypeStruct((TOKENS, EXPERTS), jnp.float32),
            jax.ShapeDtypeStruct((TOKENS, 2), jnp.int32),
            jax.ShapeDtypeStruct((TOKENS, EXPERTS), jnp.float32),
        ],
    )(x, gate_w, gate_b.reshape(1, EXPERTS))
    return (sparse, idx, logits)


# speedup vs baseline: 2.6253x; 1.0060x over previous
import jax
import jax.numpy as jnp
from jax.experimental import pallas as pl

EXPERTS = 16
HIDDEN = 2048
TOKENS = 8192
BLOCK = 2048


def _gating_body(x_ref, w_ref, b_ref, sparse_ref, idx_ref, logits_ref):
    logits = jax.lax.dot_general(
        x_ref[:], w_ref[:], (((1,), (1,)), ((), ())),
        preferred_element_type=jnp.float32,
    ) + b_ref[:]
    logits_ref[:] = logits

    col = jax.lax.broadcasted_iota(jnp.int32, logits.shape, 1)
    m1 = jnp.max(logits, axis=1, keepdims=True)
    i1 = jnp.min(jnp.where(logits == m1, col, EXPERTS), axis=1, keepdims=True)
    masked = jnp.where(col == i1, -jnp.inf, logits)
    m2 = jnp.max(masked, axis=1, keepdims=True)
    i2 = jnp.min(jnp.where(masked == m2, col, EXPERTS), axis=1, keepdims=True)

    e2 = jnp.exp(m2 - m1)
    denom = 1.0 + e2
    sparse_ref[:] = jnp.where(col == i1, 1.0 / denom,
                              jnp.where(col == i2, e2 / denom, 0.0))
    idx_ref[:] = jnp.concatenate([i1, i2], axis=1)


def kernel(x, gate_w, gate_b):
    grid = (TOKENS // BLOCK,)
    sparse, idx, logits = pl.pallas_call(
        _gating_body,
        grid=grid,
        in_specs=[
            pl.BlockSpec((BLOCK, HIDDEN), lambda i: (i, 0)),
            pl.BlockSpec((EXPERTS, HIDDEN), lambda i: (0, 0)),
            pl.BlockSpec((1, EXPERTS), lambda i: (0, 0)),
        ],
        out_specs=[
            pl.BlockSpec((BLOCK, EXPERTS), lambda i: (i, 0)),
            pl.BlockSpec((BLOCK, 2), lambda i: (i, 0)),
            pl.BlockSpec((BLOCK, EXPERTS), lambda i: (i, 0)),
        ],
        out_shape=[
            jax.ShapeDtypeStruct((TOKENS, EXPERTS), jnp.float32),
            jax.ShapeDtypeStruct((TOKENS, 2), jnp.int32),
            jax.ShapeDtypeStruct((TOKENS, EXPERTS), jnp.float32),
        ],
    )(x, gate_w, gate_b.reshape(1, EXPERTS))
    return (sparse, idx, logits)
